# baseline (speedup 1.0000x reference)
import functools

import jax
import jax.numpy as jnp
from jax import lax
from jax.experimental import pallas as pl
from jax.experimental.pallas import tpu as pltpu
from jax.experimental.pallas import tpu_sc as plsc

_D = 64
_B = 16384
_NW = 32
_BPW = _B // _NW
_SB = 256
_NC = _SB // 128
_L = 16


def _make_kernel():
    mesh = plsc.VectorSubcoreMesh(core_axis_name="c", subcore_axis_name="s")

    @functools.partial(
        pl.kernel,
        out_type=jax.ShapeDtypeStruct((_B,), jnp.float32),
        mesh=mesh,
        compiler_params=pltpu.CompilerParams(needs_layout_passes=False),
        scratch_types=[
            pltpu.VMEM((_BPW,), jnp.int32),
            pltpu.VMEM((_BPW,), jnp.int32),
            pltpu.VMEM((_BPW,), jnp.int32),
            pltpu.VMEM((_BPW,), jnp.int32),
            pltpu.VMEM((_BPW,), jnp.int32),
            pltpu.VMEM((_BPW,), jnp.int32),
            pltpu.VMEM((_SB, 128), jnp.float32),
            pltpu.VMEM((_SB, 128), jnp.float32),
            pltpu.VMEM((_SB, 128), jnp.float32),
            pltpu.VMEM((_BPW,), jnp.float32),
            pltpu.SemaphoreType.DMA,
        ],
    )
    def k(ent_hbm, heads_hbm, rels_hbm, tails_hbm, relt_hbm, out_hbm,
          hidx_v, ridx_v, tidx_v, hline_v, rline_v, tline_v,
          h_v, r_v, t_v, out_v, sem):
        wid = lax.axis_index("s") * 2 + lax.axis_index("c")
        base = wid * _BPW
        pltpu.sync_copy(heads_hbm.at[pl.ds(base, _BPW)], hidx_v)
        pltpu.sync_copy(rels_hbm.at[pl.ds(base, _BPW)], ridx_v)
        pltpu.sync_copy(tails_hbm.at[pl.ds(base, _BPW)], tidx_v)

        def shift(g, _):
            sl = pl.ds(g * _L, _L)
            hline_v[sl] = jnp.right_shift(hidx_v[sl], 1)
            rline_v[sl] = jnp.right_shift(ridx_v[sl], 1)
            tline_v[sl] = jnp.right_shift(tidx_v[sl], 1)
            return 0

        lax.fori_loop(0, _BPW // _L, shift, 0)

        lane = lax.iota(jnp.int32, _L)

        def half(sb, _):
            sb0 = sb * _SB
            cps = []
            for c in range(_NC):
                isl = pl.ds(sb0 + c * 128, 128)
                dsl = pl.ds(c * 128, 128)
                cps.append(pltpu.async_copy(
                    ent_hbm.at[hline_v.at[isl]], h_v.at[dsl], sem))
                cps.append(pltpu.async_copy(
                    relt_hbm.at[rline_v.at[isl]], r_v.at[dsl], sem))
                cps.append(pltpu.async_copy(
                    ent_hbm.at[tline_v.at[isl]], t_v.at[dsl], sem))
            for cp in cps:
                cp.wait()

            def group(g, _):
                rows = g * _L + lane
                isl = pl.ds(sb0 + g * _L, _L)
                hoff = jnp.bitwise_and(hidx_v[isl], 1) * _D
                roff = jnp.bitwise_and(ridx_v[isl], 1) * _D
                toff = jnp.bitwise_and(tidx_v[isl], 1) * _D
                acc = jnp.zeros((_L,), jnp.float32)
                for d in range(_D):
                    acc = acc + (
                        plsc.load_gather(h_v, [rows, hoff + d])
                        * plsc.load_gather(r_v, [rows, roff + d])
                        * plsc.load_gather(t_v, [rows, toff + d]))
                out_v[pl.ds(sb0 + g * _L, _L)] = acc
                return 0

            lax.fori_loop(0, _SB // _L, group, 0)
            return 0

        lax.fori_loop(0, _BPW // _SB, half, 0)
        pltpu.sync_copy(out_v, out_hbm.at[pl.ds(base, _BPW)])

    return k


_kernel_call = _make_kernel()


def kernel(entity_emb, heads, relations, tails, rel_table):
    ent2 = jnp.reshape(entity_emb, (entity_emb.shape[0] // 2, 2 * _D))
    rel2 = jnp.reshape(rel_table, (rel_table.shape[0] // 2, 2 * _D))
    return _kernel_call(
        ent2,
        heads.astype(jnp.int32),
        relations.astype(jnp.int32),
        tails.astype(jnp.int32),
        rel2,
    )


# speedup vs baseline: 1.5882x; 1.5882x over previous
import functools

import jax
import jax.numpy as jnp
from jax import lax
from jax.experimental import pallas as pl
from jax.experimental.pallas import tpu as pltpu
from jax.experimental.pallas import tpu_sc as plsc

_D = 64
_B = 16384
_NW = 32
_BPW = _B // _NW
_SB = 256
_NC = _SB // 128
_L = 16

_NE = 1000000
_BC = 4096
_TGRID = (_NE + _BC - 1) // _BC


def _tc_transpose(ent_t):
    def body(x_ref, o_ref):
        x = x_ref[...]
        eye = jax.lax.broadcasted_iota(jnp.int32, (_D, _D), 0)
        eyec = jax.lax.broadcasted_iota(jnp.int32, (_D, _D), 1)
        ident = jnp.where(eye == eyec, 1.0, 0.0).astype(jnp.float32)
        y = jax.lax.dot_general(
            x, ident, (((0,), (0,)), ((), ())),
            preferred_element_type=jnp.float32)
        o_ref[...] = jnp.concatenate(
            [y, jnp.zeros((_BC, 128 - _D), jnp.float32)], axis=1)

    return pl.pallas_call(
        body,
        grid=(_TGRID,),
        in_specs=[pl.BlockSpec((_D, _BC), lambda i: (0, i))],
        out_specs=pl.BlockSpec((_BC, 128), lambda i: (i, 0)),
        out_shape=jax.ShapeDtypeStruct((_NE, 128), jnp.float32),
    )(ent_t)


def _make_sc_kernel():
    mesh = plsc.VectorSubcoreMesh(core_axis_name="c", subcore_axis_name="s")

    @functools.partial(
        pl.kernel,
        out_type=jax.ShapeDtypeStruct((_B,), jnp.float32),
        mesh=mesh,
        compiler_params=pltpu.CompilerParams(needs_layout_passes=False),
        scratch_types=[
            pltpu.VMEM((_BPW,), jnp.int32),
            pltpu.VMEM((_BPW,), jnp.int32),
            pltpu.VMEM((_BPW,), jnp.int32),
            pltpu.VMEM((_BPW,), jnp.int32),
            pltpu.VMEM((_SB, 128), jnp.float32),
            pltpu.VMEM((_SB, 128), jnp.float32),
            pltpu.VMEM((_SB, 128), jnp.float32),
            pltpu.VMEM((_BPW,), jnp.float32),
            pltpu.SemaphoreType.DMA,
        ],
    )
    def k(ent_hbm, heads_hbm, rels_hbm, tails_hbm, relt_hbm, out_hbm,
          hidx_v, ridx_v, tidx_v, rline_v, h_v, r_v, t_v, out_v, sem):
        wid = lax.axis_index("s") * 2 + lax.axis_index("c")
        base = wid * _BPW
        pltpu.sync_copy(heads_hbm.at[pl.ds(base, _BPW)], hidx_v)
        pltpu.sync_copy(rels_hbm.at[pl.ds(base, _BPW)], ridx_v)
        pltpu.sync_copy(tails_hbm.at[pl.ds(base, _BPW)], tidx_v)

        def shift(g, _):
            sl = pl.ds(g * _L, _L)
            rline_v[sl] = jnp.right_shift(ridx_v[sl], 1)
            return 0

        lax.fori_loop(0, _BPW // _L, shift, 0)

        lane = lax.iota(jnp.int32, _L)

        def half(sb, _):
            sb0 = sb * _SB
            cps = []
            for c in range(_NC):
                isl = pl.ds(sb0 + c * 128, 128)
                dsl = pl.ds(c * 128, 128)
                cps.append(pltpu.async_copy(
                    ent_hbm.at[hidx_v.at[isl]], h_v.at[dsl], sem))
                cps.append(pltpu.async_copy(
                    relt_hbm.at[rline_v.at[isl]], r_v.at[dsl], sem))
                cps.append(pltpu.async_copy(
                    ent_hbm.at[tidx_v.at[isl]], t_v.at[dsl], sem))
            for cp in cps:
                cp.wait()

            def group(g, _):
                rows = g * _L + lane
                isl = pl.ds(sb0 + g * _L, _L)
                roff = jnp.bitwise_and(ridx_v[isl], 1) * _D
                acc = jnp.zeros((_L,), jnp.float32)
                for d in range(_D):
                    col = jnp.full((_L,), d, jnp.int32)
                    acc = acc + (
                        plsc.load_gather(h_v, [rows, col])
                        * plsc.load_gather(r_v, [rows, roff + d])
                        * plsc.load_gather(t_v, [rows, col]))
                out_v[pl.ds(sb0 + g * _L, _L)] = acc
                return 0

            lax.fori_loop(0, _SB // _L, group, 0)
            return 0

        lax.fori_loop(0, _BPW // _SB, half, 0)
        pltpu.sync_copy(out_v, out_hbm.at[pl.ds(base, _BPW)])

    return k


_sc_call = _make_sc_kernel()


def kernel(entity_emb, heads, relations, tails, rel_table):
    ent_pad = _tc_transpose(entity_emb.T)
    rel2 = jnp.reshape(rel_table, (rel_table.shape[0] // 2, 2 * _D))
    return _sc_call(
        ent_pad,
        heads.astype(jnp.int32),
        relations.astype(jnp.int32),
        tails.astype(jnp.int32),
        rel2,
    )
